# Initial kernel scaffold; baseline (speedup 1.0000x reference)
import jax
import jax.numpy as jnp
from jax.experimental import pallas as pl


def kernel(hidden_states, attention_mask, reason_token_mask, Wq, bq, Wk, bk, Wv, bv, Wo, bo, ln_g, ln_b, W1, b1, W2, b2, start_emb, end_emb):
    raise NotImplementedError("write your pallas kernel here")


# speedup vs baseline: 16.2782x; 16.2782x over previous
import functools
import math

import jax
import jax.numpy as jnp
from jax.experimental import pallas as pl

H = 768
NH = 12
HD = H // NH

TOK_BLOCK = 256


def _fused_kernel(x_ref, wqkv_ref, bqkv_ref, wo_ref, bo_ref, lng_ref, lnb_ref,
                  w1_ref, b1_ref, w2_ref, b2_ref, markers_ref, out_ref, *, T):
    i = pl.program_id(0)
    x = x_ref[...]

    qkv = jnp.dot(x, wqkv_ref[...], preferred_element_type=jnp.float32) + bqkv_ref[...]
    mqkv = jnp.dot(markers_ref[...], wqkv_ref[...],
                   preferred_element_type=jnp.float32) + bqkv_ref[...]
    q = qkv[:, :H]
    k = qkv[:, H:2 * H]
    v = qkv[:, 2 * H:]
    ks = mqkv[0:1, H:2 * H]
    ke = mqkv[1:2, H:2 * H]
    vs = mqkv[0:1, 2 * H:]
    ve = mqkv[1:2, 2 * H:]

    rows = jax.lax.broadcasted_iota(jnp.int32, (H, NH), 0)
    cols = jax.lax.broadcasted_iota(jnp.int32, (H, NH), 1)
    sel = (rows // HD == cols).astype(jnp.float32)
    inv_sqrt_hd = 1.0 / math.sqrt(HD)

    s_st = jnp.dot(q * ks, sel, preferred_element_type=jnp.float32) * inv_sqrt_hd
    s_md = jnp.dot(q * k, sel, preferred_element_type=jnp.float32) * inv_sqrt_hd
    s_en = jnp.dot(q * ke, sel, preferred_element_type=jnp.float32) * inv_sqrt_hd

    m = jnp.maximum(jnp.maximum(s_st, s_md), s_en)
    e0 = jnp.exp(s_st - m)
    e1 = jnp.exp(s_md - m)
    e2 = jnp.exp(s_en - m)
    denom = e0 + e1 + e2
    w0 = e0 / denom
    w1w = e1 / denom
    w2w = e2 / denom

    selT = sel.T
    o = (jnp.dot(w0, selT, preferred_element_type=jnp.float32) * vs
         + jnp.dot(w1w, selT, preferred_element_type=jnp.float32) * v
         + jnp.dot(w2w, selT, preferred_element_type=jnp.float32) * ve)

    attn = jnp.dot(o, wo_ref[...], preferred_element_type=jnp.float32) + bo_ref[...]

    mu = jnp.mean(attn, axis=-1, keepdims=True)
    var = jnp.mean((attn - mu) ** 2, axis=-1, keepdims=True)
    y = (attn - mu) / jnp.sqrt(var + 1e-5) * lng_ref[...] + lnb_ref[...]

    flat = jax.lax.broadcasted_iota(jnp.int32, (TOK_BLOCK, 1), 0) + i * TOK_BLOCK
    passthru = jnp.logical_or(flat == 0, flat % T == T - 1)
    processed = jnp.where(passthru, x, y)

    h1 = jnp.dot(processed, w1_ref[...], preferred_element_type=jnp.float32) + b1_ref[...]
    g = 0.5 * h1 * (1.0 + jax.lax.erf(h1 * (1.0 / math.sqrt(2.0))))
    agg = jnp.dot(g, w2_ref[...], preferred_element_type=jnp.float32) + b2_ref[...]
    out_ref[...] = x + agg


def kernel(hidden_states, attention_mask, reason_token_mask, Wq, bq, Wk, bk,
           Wv, bv, Wo, bo, ln_g, ln_b, W1, b1, W2, b2, start_emb, end_emb):
    B, T, Hs = hidden_states.shape
    N = B * T
    x = hidden_states.reshape(N, Hs)

    Wqkv = jnp.concatenate([Wq, Wk, Wv], axis=1)
    bqkv = jnp.concatenate([bq, bk, bv]).reshape(1, 3 * Hs)
    markers = jnp.stack([start_emb, end_emb], axis=0)

    grid = (N // TOK_BLOCK,)
    full = lambda a: pl.BlockSpec(a.shape, lambda i: (0,) * a.ndim)
    out = pl.pallas_call(
        functools.partial(_fused_kernel, T=T),
        grid=grid,
        in_specs=[
            pl.BlockSpec((TOK_BLOCK, Hs), lambda i: (i, 0)),
            full(Wqkv),
            full(bqkv),
            full(Wo),
            full(bo.reshape(1, Hs)),
            full(ln_g.reshape(1, Hs)),
            full(ln_b.reshape(1, Hs)),
            full(W1),
            full(b1.reshape(1, 2 * Hs)),
            full(W2),
            full(b2.reshape(1, Hs)),
            full(markers),
        ],
        out_specs=pl.BlockSpec((TOK_BLOCK, Hs), lambda i: (i, 0)),
        out_shape=jax.ShapeDtypeStruct((N, Hs), jnp.float32),
    )(x, Wqkv, bqkv, Wo, bo.reshape(1, Hs), ln_g.reshape(1, Hs),
      ln_b.reshape(1, Hs), W1, b1.reshape(1, 2 * Hs), W2, b2.reshape(1, Hs),
      markers)
    return out.reshape(B, T, Hs)
